# baseline (speedup 1.0000x reference)
import jax
import jax.numpy as jnp
from jax import lax
from jax.experimental import pallas as pl
from jax.experimental.pallas import tpu as pltpu
from jax.experimental.pallas import tpu_sc as plsc

NUM_CORES = 2
NUM_SUBCORES = 16
LANES = 16
NW = NUM_CORES * NUM_SUBCORES

BATCH = 16384
EMBED_DIM = 64
B_PER_W = BATCH // NW
CHUNK = 128
NCHUNK = B_PER_W // CHUNK
GPC = CHUNK // LANES


def _mesh():
    return plsc.VectorSubcoreMesh(
        core_axis_name="c", subcore_axis_name="s",
        num_cores=NUM_CORES, num_subcores=NUM_SUBCORES)


def _dot_kernel(user_input, movie_input, user_emb2, movie_emb2, out_hbm,
                idx_u, idx_m, pidx_u, pidx_m, rows_u, rows_m, out_v,
                sem_a, sem_b, sem_i):
    wid = lax.axis_index("s") * NUM_CORES + lax.axis_index("c")
    base = wid * B_PER_W

    icopies = []
    for j in range(NCHUNK):
        off = base + j * CHUNK
        icopies.append(pltpu.async_copy(user_input.at[pl.ds(off, CHUNK)], idx_u.at[j], sem_i))
        icopies.append(pltpu.async_copy(movie_input.at[pl.ds(off, CHUNK)], idx_m.at[j], sem_i))
    for c in icopies:
        c.wait()
    for j in range(NCHUNK):
        for k in range(GPC):
            s = pl.ds(k * LANES, LANES)
            pidx_u[j, s] = idx_u[j, s] >> 1
            pidx_m[j, s] = idx_m[j, s] >> 1

    sems = [sem_a, sem_b]
    lane = lax.iota(jnp.int32, LANES)

    def fire(j):
        b = j % 2
        return [
            pltpu.async_copy(user_emb2.at[pidx_u.at[j]], rows_u.at[b], sems[b]),
            pltpu.async_copy(movie_emb2.at[pidx_m.at[j]], rows_m.at[b], sems[b]),
        ]

    inflight = {0: fire(0), 1: fire(1)}
    for j in range(NCHUNK):
        b = j % 2
        for c in inflight.pop(j):
            c.wait()
        jvec = jnp.full((LANES,), j, jnp.int32)
        bvec = jnp.full((LANES,), b, jnp.int32)

        def group_body(k, carry):
            rows = k * LANES + lane
            iu = plsc.load_gather(idx_u, [jvec, rows])
            im = plsc.load_gather(idx_m, [jvec, rows])
            offu = (iu & 1) << 6
            offm = (im & 1) << 6
            acc = jnp.zeros((LANES,), jnp.float32)
            for d in range(EMBED_DIM):
                u = plsc.load_gather(rows_u, [bvec, rows, offu + d])
                m = plsc.load_gather(rows_m, [bvec, rows, offm + d])
                acc = acc + u * m
            plsc.store_scatter(out_v, [j * CHUNK + rows], acc)
            return carry

        lax.fori_loop(0, GPC, group_body, 0)
        if j + 2 < NCHUNK:
            inflight[j + 2] = fire(j + 2)

    pltpu.sync_copy(out_v, out_hbm.at[pl.ds(base, B_PER_W)])


def _bias_kernel(dot_hbm, user_input, movie_input, bias_u16, bias_m16,
                 out_hbm, dot_v, idx_u, idx_m, bidx_u, bidx_m,
                 bias_u, bias_m, out_v, sem):
    wid = lax.axis_index("s") * NUM_CORES + lax.axis_index("c")
    base = wid * B_PER_W

    copies = [pltpu.async_copy(dot_hbm.at[pl.ds(base, B_PER_W)], dot_v, sem)]
    for j in range(NCHUNK):
        off = base + j * CHUNK
        copies.append(pltpu.async_copy(user_input.at[pl.ds(off, CHUNK)], idx_u.at[j], sem))
        copies.append(pltpu.async_copy(movie_input.at[pl.ds(off, CHUNK)], idx_m.at[j], sem))
    for c in copies:
        c.wait()
    for j in range(NCHUNK):
        for k in range(GPC):
            s = pl.ds(k * LANES, LANES)
            bidx_u[j, s] = idx_u[j, s] >> 4
            bidx_m[j, s] = idx_m[j, s] >> 4

    bcopies = []
    for j in range(NCHUNK):
        bcopies.append(pltpu.async_copy(bias_u16.at[bidx_u.at[j]], bias_u.at[j], sem))
        bcopies.append(pltpu.async_copy(bias_m16.at[bidx_m.at[j]], bias_m.at[j], sem))
    for c in bcopies:
        c.wait()

    lane = lax.iota(jnp.int32, LANES)

    def group_body(g, carry):
        j = g // GPC
        jvec = jnp.full((LANES,), j, jnp.int32)
        rows = (g % GPC) * LANES + lane
        flat = g * LANES + lane
        iu = plsc.load_gather(idx_u, [jvec, rows])
        im = plsc.load_gather(idx_m, [jvec, rows])
        bu = plsc.load_gather(bias_u, [jvec, rows, iu & 15])
        bm = plsc.load_gather(bias_m, [jvec, rows, im & 15])
        x = plsc.load_gather(dot_v, [flat]) + bu + bm
        y = 1.0 / (1.0 + jnp.exp(-x))
        plsc.store_scatter(out_v, [flat], y)
        return carry

    lax.fori_loop(0, NCHUNK * GPC, group_body, 0)

    pltpu.sync_copy(out_v, out_hbm.at[pl.ds(base, B_PER_W)])


def kernel(user_input, movie_input, user_emb, user_bias, movie_emb, movie_bias):
    dot_f = pl.kernel(
        _dot_kernel,
        mesh=_mesh(),
        compiler_params=pltpu.CompilerParams(
            needs_layout_passes=False, use_tc_tiling_on_sc=True),
        out_type=jax.ShapeDtypeStruct((BATCH,), jnp.float32),
        scratch_types=[
            pltpu.VMEM((NCHUNK, CHUNK), jnp.int32),
            pltpu.VMEM((NCHUNK, CHUNK), jnp.int32),
            pltpu.VMEM((NCHUNK, CHUNK), jnp.int32),
            pltpu.VMEM((NCHUNK, CHUNK), jnp.int32),
            pltpu.VMEM((2, CHUNK, 2 * EMBED_DIM), jnp.float32),
            pltpu.VMEM((2, CHUNK, 2 * EMBED_DIM), jnp.float32),
            pltpu.VMEM((B_PER_W,), jnp.float32),
            pltpu.SemaphoreType.DMA,
            pltpu.SemaphoreType.DMA,
            pltpu.SemaphoreType.DMA,
        ],
    )
    bias_f = pl.kernel(
        _bias_kernel,
        mesh=_mesh(),
        compiler_params=pltpu.CompilerParams(
            needs_layout_passes=False, use_tc_tiling_on_sc=False),
        out_type=jax.ShapeDtypeStruct((BATCH,), jnp.float32),
        scratch_types=[
            pltpu.VMEM((B_PER_W,), jnp.float32),
            pltpu.VMEM((NCHUNK, CHUNK), jnp.int32),
            pltpu.VMEM((NCHUNK, CHUNK), jnp.int32),
            pltpu.VMEM((NCHUNK, CHUNK), jnp.int32),
            pltpu.VMEM((NCHUNK, CHUNK), jnp.int32),
            pltpu.VMEM((NCHUNK, CHUNK, LANES), jnp.float32),
            pltpu.VMEM((NCHUNK, CHUNK, LANES), jnp.float32),
            pltpu.VMEM((B_PER_W,), jnp.float32),
            pltpu.SemaphoreType.DMA,
        ],
    )
    ue2 = user_emb.reshape(user_emb.shape[0] // 2, 2 * EMBED_DIM)
    me2 = movie_emb.reshape(movie_emb.shape[0] // 2, 2 * EMBED_DIM)
    ub16 = user_bias.reshape(user_bias.shape[0] // LANES, LANES)
    mb16 = movie_bias.reshape(movie_bias.shape[0] // LANES, LANES)
    dot = dot_f(user_input, movie_input, ue2, me2)
    return bias_f(dot, user_input, movie_input, ub16, mb16)
